# baseline (speedup 1.0000x reference)
import functools

import jax
import jax.numpy as jnp
from jax import lax
from jax.experimental import pallas as pl
from jax.experimental.pallas import tpu as pltpu
from jax.experimental.pallas import tpu_sc as plsc

N = 10000
E = 320000
D = 128
G = 64

_NC = 2
_NS = 16
_NW = _NC * _NS
_EPW = E // _NW
_CH = 80
_NCHUNK = _EPW // _CH
_NP = 10240
_RPT = _NP // _NS
_DEGW = 16
_PCH = 80
_PNC = N // _PCH

_BLK = 1000
_GRID = N // _BLK


def _sc_mesh():
    return plsc.VectorSubcoreMesh(core_axis_name="c", subcore_axis_name="s")


_SC_PARAMS = pltpu.CompilerParams(use_tc_tiling_on_sc=False,
                                  needs_layout_passes=False)


def _deg_body(dst3_hbm, ones_hbm, zeros_hbm, out_hbm, acc, ones_v, dstw_v,
              drain_v, sem):
    cid = lax.axis_index("c")
    tid = lax.axis_index("s")
    wid = cid * _NS + tid
    pltpu.sync_copy(zeros_hbm, drain_v)
    pltpu.sync_copy(drain_v, acc.at[pl.ds(tid * _RPT, _RPT)])
    pltpu.sync_copy(ones_hbm, ones_v)
    pltpu.sync_copy(dst3_hbm.at[wid], dstw_v)
    plsc.subcore_barrier()

    k = 25
    for grp in range(_NCHUNK // k):
        def fire(j, carry):
            pltpu.async_copy(ones_v, acc.at[dstw_v.at[j]], sem, add=True)
            return carry

        def drain(j, carry):
            pltpu.make_async_copy(ones_v, acc.at[dstw_v.at[j]], sem).wait()
            return carry

        lax.fori_loop(grp * k, (grp + 1) * k, fire, 0)
        lax.fori_loop(grp * k, (grp + 1) * k, drain, 0)

    plsc.subcore_barrier()
    pltpu.sync_copy(acc.at[pl.ds(tid * _RPT, _RPT)], drain_v)
    pltpu.sync_copy(drain_v, out_hbm.at[cid, pl.ds(tid * _RPT, _RPT)])


_deg_call = functools.partial(
    pl.kernel,
    compiler_params=_SC_PARAMS,
    out_type=jax.ShapeDtypeStruct((_NC, _NP, _DEGW), jnp.float32),
    mesh=_sc_mesh(),
    scratch_types=[
        pltpu.VMEM_SHARED((_NP, _DEGW), jnp.float32),
        pltpu.VMEM((_CH, _DEGW), jnp.float32),
        pltpu.VMEM((_NCHUNK, _CH), jnp.int32),
        pltpu.VMEM((_RPT, _DEGW), jnp.float32),
        pltpu.SemaphoreType.DMA,
    ],
)(_deg_body)


def _conv_body(t_hbm, src_hbm, dst3_hbm, zeros_hbm, out_hbm, acc, src_v,
               dstw_v, rows0, rows1, semg0, semg1, sems0, sems1):
    cid = lax.axis_index("c")
    tid = lax.axis_index("s")
    wid = cid * _NS + tid
    pltpu.sync_copy(zeros_hbm, rows0)
    for k in range(_RPT // _CH):
        pltpu.sync_copy(rows0, acc.at[pl.ds(tid * _RPT + k * _CH, _CH)])
    base = wid * _EPW
    pltpu.sync_copy(src_hbm.at[pl.ds(base, _EPW)], src_v)
    pltpu.sync_copy(dst3_hbm.at[wid], dstw_v)
    plsc.subcore_barrier()

    def gstart(j, rows, sem):
        pltpu.async_copy(t_hbm.at[src_v.at[pl.ds(j * _CH, _CH)]], rows, sem)

    def gwait(j, rows, sem):
        pltpu.make_async_copy(t_hbm.at[src_v.at[pl.ds(j * _CH, _CH)]],
                              rows, sem).wait()

    def sstart(j, rows, sem):
        pltpu.async_copy(rows, acc.at[dstw_v.at[j]], sem, add=True)

    def swait(j, rows, sem):
        pltpu.make_async_copy(rows, acc.at[dstw_v.at[j]], sem).wait()

    npair = (_NCHUNK - 1) // 2
    gstart(0, rows0, semg0)
    gstart(1, rows1, semg1)

    def pair(m, carry):
        j0 = 2 * m
        gwait(j0, rows0, semg0)
        sstart(j0, rows0, sems0)
        gwait(j0 + 1, rows1, semg1)
        sstart(j0 + 1, rows1, sems1)
        swait(j0, rows0, sems0)

        @pl.when(m < npair - 1)
        def _():
            gstart(j0 + 2, rows0, semg0)

        swait(j0 + 1, rows1, sems1)

        @pl.when(m < npair - 1)
        def _():
            gstart(j0 + 3, rows1, semg1)

        return carry

    lax.fori_loop(0, npair, pair, 0)
    jt = _NCHUNK - 1
    gstart(jt, rows0, semg0)
    gwait(jt, rows0, semg0)
    pltpu.sync_copy(rows0, acc.at[dstw_v.at[jt]], add=True)
    plsc.subcore_barrier()
    for k in range(_RPT // _CH):
        rb = rows0 if k % 2 == 0 else rows1
        pltpu.sync_copy(acc.at[pl.ds(tid * _RPT + k * _CH, _CH)], rb)
        pltpu.sync_copy(rb, out_hbm.at[cid, pl.ds(tid * _RPT + k * _CH, _CH)])


_conv_call = functools.partial(
    pl.kernel,
    compiler_params=_SC_PARAMS,
    out_type=jax.ShapeDtypeStruct((_NC, _NP, D), jnp.float32),
    mesh=_sc_mesh(),
    scratch_types=[
        pltpu.VMEM_SHARED((_NP, D), jnp.float32),
        pltpu.VMEM((_EPW,), jnp.int32),
        pltpu.VMEM((_NCHUNK, _CH), jnp.int32),
        pltpu.VMEM((_CH, D), jnp.float32),
        pltpu.VMEM((_CH, D), jnp.float32),
        pltpu.SemaphoreType.DMA,
        pltpu.SemaphoreType.DMA,
        pltpu.SemaphoreType.DMA,
        pltpu.SemaphoreType.DMA,
    ],
)(_conv_body)


def _pool_body(h_hbm, batch_hbm, ninf_hbm, zsum_hbm, zcnt_hbm,
               sum_out, max_out, cnt_out,
               rbuf, bbuf, sumacc, maxacc, cntacc):
    cid = lax.axis_index("c")
    tid = lax.axis_index("s")
    wid = cid * _NS + tid
    pltpu.sync_copy(ninf_hbm, maxacc)
    pltpu.sync_copy(zsum_hbm, sumacc)
    pltpu.sync_copy(zcnt_hbm, cntacc)

    lane = lax.broadcasted_iota(jnp.int32, (16,), 0)
    lane0 = lane == 0
    zero16 = jnp.zeros((16,), jnp.int32)
    dnums = lax.GatherDimensionNumbers(
        offset_dims=(), collapsed_slice_dims=(0,), start_index_map=(0,))

    def do_chunk(k, carry):
        cidx = wid + _NW * k
        row0 = cidx * _PCH
        pltpu.sync_copy(h_hbm.at[pl.ds(row0, _PCH)], rbuf)
        pltpu.sync_copy(batch_hbm.at[pl.ds(row0, _PCH)], bbuf)

        def do_sub(b, carry2):
            bvec = bbuf[pl.ds(b * 16, 16)]

            def do_lane(l, carry3):
                idx = jnp.full((16, 1), 0, jnp.int32) + l
                g16 = lax.gather(bvec, idx, dnums, (1,),
                                 mode=lax.GatherScatterMode.PROMISE_IN_BOUNDS)
                r = b * 16 + l
                for c in range(D // 16):
                    col = lane + c * 16
                    v = rbuf[r, pl.ds(c * 16, 16)]
                    cur = plsc.load_gather(maxacc, [g16, col])
                    plsc.store_scatter(maxacc, [g16, col], jnp.maximum(cur, v))
                    cur2 = plsc.load_gather(sumacc, [g16, col])
                    plsc.store_scatter(sumacc, [g16, col], cur2 + v)
                cnt = plsc.load_gather(cntacc, [zero16, g16])
                plsc.store_scatter(cntacc, [zero16, g16], cnt + 1.0, mask=lane0)
                return carry3

            return lax.fori_loop(0, 16, do_lane, carry2)

        lax.fori_loop(0, _PCH // 16, do_sub, carry)
        return carry

    nch = jnp.where(wid < _PNC - 3 * _NW, 4, 3)
    lax.fori_loop(0, nch, do_chunk, 0)
    pltpu.sync_copy(sumacc, sum_out.at[wid])
    pltpu.sync_copy(maxacc, max_out.at[wid])
    pltpu.sync_copy(cntacc, cnt_out.at[wid])


_pool_call = functools.partial(
    pl.kernel,
    compiler_params=_SC_PARAMS,
    out_type=(
        jax.ShapeDtypeStruct((_NW, G, D), jnp.float32),
        jax.ShapeDtypeStruct((_NW, G, D), jnp.float32),
        jax.ShapeDtypeStruct((_NW, 8, G), jnp.float32),
    ),
    mesh=_sc_mesh(),
    scratch_types=[
        pltpu.VMEM((_PCH, D), jnp.float32),
        pltpu.VMEM((_PCH,), jnp.int32),
        pltpu.VMEM((G, D), jnp.float32),
        pltpu.VMEM((G, D), jnp.float32),
        pltpu.VMEM((8, G), jnp.float32),
    ],
)(_pool_body)


def _pre1_body(x_ref, degp_ref, w_ref, t_ref, dinv_ref):
    dp = degp_ref[...]
    deg = dp[0, :, 0:1] + dp[1, :, 0:1] + 1.0
    dinv = 1.0 / jnp.sqrt(deg)
    dinv_ref[...] = dinv
    t_ref[...] = jnp.dot(x_ref[...], w_ref[...],
                         preferred_element_type=jnp.float32) * dinv


def _pre1_call(x, degp, w):
    return pl.pallas_call(
        _pre1_body,
        grid=(_GRID,),
        in_specs=[
            pl.BlockSpec((_BLK, D), lambda i: (i, 0)),
            pl.BlockSpec((_NC, _BLK, _DEGW), lambda i: (0, i, 0)),
            pl.BlockSpec((D, D), lambda i: (0, 0)),
        ],
        out_specs=[
            pl.BlockSpec((_BLK, D), lambda i: (i, 0)),
            pl.BlockSpec((_BLK, 1), lambda i: (i, 0)),
        ],
        out_shape=[
            jax.ShapeDtypeStruct((N, D), jnp.float32),
            jax.ShapeDtypeStruct((N, 1), jnp.float32),
        ],
    )(x, degp, w)


def _make_post_body(relu_ln, has_res, has_next):
    def body(*refs):
        refs = list(refs)
        conv_ref = refs.pop(0)
        t_ref = refs.pop(0)
        dinv_ref = refs.pop(0)
        b_ref = refs.pop(0)
        gamma_ref = refs.pop(0) if relu_ln else None
        beta_ref = refs.pop(0) if relu_ln else None
        res_ref = refs.pop(0) if has_res else None
        w_ref = refs.pop(0) if has_next else None
        h_ref = refs.pop(0)
        tn_ref = refs.pop(0) if has_next else None

        dinv = dinv_ref[...]
        s = (conv_ref[0] + conv_ref[1] + t_ref[...]) * dinv + b_ref[...]
        if relu_ln:
            a = jnp.maximum(s, 0.0)
            mu = jnp.mean(a, axis=-1, keepdims=True)
            var = jnp.mean((a - mu) ** 2, axis=-1, keepdims=True)
            h = (a - mu) / jnp.sqrt(var + 1e-5) * gamma_ref[...] + beta_ref[...]
        else:
            h = s
        if has_res:
            h = h + res_ref[...]
        h_ref[...] = h
        if has_next:
            tn_ref[...] = jnp.dot(h, w_ref[...],
                                  preferred_element_type=jnp.float32) * dinv
    return body


def _post_call(convp, t, dinv, b, gamma, beta, res, w_next, relu_ln):
    has_res = res is not None
    has_next = w_next is not None
    in_specs = [
        pl.BlockSpec((_NC, _BLK, D), lambda i: (0, i, 0)),
        pl.BlockSpec((_BLK, D), lambda i: (i, 0)),
        pl.BlockSpec((_BLK, 1), lambda i: (i, 0)),
        pl.BlockSpec((1, D), lambda i: (0, 0)),
    ]
    args = [convp, t, dinv, b.reshape(1, D)]
    if relu_ln:
        in_specs += [pl.BlockSpec((1, D), lambda i: (0, 0))] * 2
        args += [gamma.reshape(1, D), beta.reshape(1, D)]
    if has_res:
        in_specs.append(pl.BlockSpec((_BLK, D), lambda i: (i, 0)))
        args.append(res)
    if has_next:
        in_specs.append(pl.BlockSpec((D, D), lambda i: (0, 0)))
        args.append(w_next)
    out_specs = [pl.BlockSpec((_BLK, D), lambda i: (i, 0))]
    out_shape = [jax.ShapeDtypeStruct((N, D), jnp.float32)]
    if has_next:
        out_specs.append(pl.BlockSpec((_BLK, D), lambda i: (i, 0)))
        out_shape.append(jax.ShapeDtypeStruct((N, D), jnp.float32))
    res_out = pl.pallas_call(
        _make_post_body(relu_ln, has_res, has_next),
        grid=(_GRID,),
        in_specs=in_specs,
        out_specs=out_specs,
        out_shape=out_shape,
    )(*args)
    return res_out if has_next else res_out[0]


def _head_body(sum_ref, max_ref, cnt_ref, w1_ref, b1_ref, w2_ref, b2_ref,
               out_ref):
    cnt = cnt_ref[0, 0]
    for i in range(1, _NW):
        cnt = cnt + cnt_ref[i, 0]
    s = sum_ref[0]
    m = max_ref[0]
    for i in range(1, _NW):
        s = s + sum_ref[i]
        m = jnp.maximum(m, max_ref[i])
    counts = jnp.maximum(cnt, 1.0)[:, None]
    mean = s / counts
    m = jnp.where(m == -jnp.inf, 0.0, m)
    gcat = jnp.concatenate([mean, m], axis=1)
    a = jnp.maximum(
        jnp.dot(gcat, w1_ref[...], preferred_element_type=jnp.float32)
        + b1_ref[...], 0.0)
    out_ref[...] = jnp.dot(a, w2_ref[...],
                           preferred_element_type=jnp.float32) + b2_ref[...]


def _head_call(sump, maxp, cntp, w1, b1, w2, b2):
    nout = w2.shape[1]
    return pl.pallas_call(
        _head_body,
        out_shape=jax.ShapeDtypeStruct((G, nout), jnp.float32),
    )(sump, maxp, cntp, w1, b1.reshape(1, D), w2, b2.reshape(1, nout))


def kernel(x, edge_index, batch, W_in, b_in, W1, b1, W2, b2, W3, b3,
           W_out, b_out, gamma, beta, fc1_W, fc1_b, fc2_W, fc2_b):
    f32 = jnp.float32
    src = edge_index[0]
    dst3 = edge_index[1].reshape(_NW, _NCHUNK, _CH)

    zeros_deg = jnp.zeros((_RPT, _DEGW), f32)
    ones_deg = jnp.ones((_CH, _DEGW), f32)
    zeros_conv = jnp.zeros((_CH, D), f32)
    ninf = jnp.full((G, D), -jnp.inf, f32)
    zsum = jnp.zeros((G, D), f32)
    zcnt = jnp.zeros((8, G), f32)

    degp = _deg_call(dst3, ones_deg, zeros_deg)

    t, dinv = _pre1_call(x, degp, W_in)

    convp = _conv_call(t, src, dst3, zeros_conv)
    h, t = _post_call(convp, t, dinv, b_in, gamma, beta, None, W1,
                      relu_ln=True)
    for (b_cur, w_next) in [(b1, W2), (b2, W3), (b3, W_out)]:
        convp = _conv_call(t, src, dst3, zeros_conv)
        h, t = _post_call(convp, t, dinv, b_cur, gamma, beta, h, w_next,
                          relu_ln=True)
    convp = _conv_call(t, src, dst3, zeros_conv)
    h_out = _post_call(convp, t, dinv, b_out, None, None, None, None,
                       relu_ln=False)

    sump, maxp, cntp = _pool_call(h_out, batch, ninf, zsum, zcnt)

    return _head_call(sump, maxp, cntp, fc1_W, fc1_b, fc2_W, fc2_b)


# speedup vs baseline: 1.2210x; 1.2210x over previous
import functools

import jax
import jax.numpy as jnp
from jax import lax
from jax.experimental import pallas as pl
from jax.experimental.pallas import tpu as pltpu
from jax.experimental.pallas import tpu_sc as plsc

N = 10000
E = 320000
D = 128
G = 64

_NC = 2
_NS = 16
_NW = _NC * _NS
_EPW = E // _NW
_CH = 80
_NCHUNK = _EPW // _CH
_NP = 10240
_RPT = _NP // _NS
_DEGW = 16
_PCH = 80
_PNC = N // _PCH

_BLK = 1000
_GRID = N // _BLK


def _sc_mesh():
    return plsc.VectorSubcoreMesh(core_axis_name="c", subcore_axis_name="s")


_SC_PARAMS = pltpu.CompilerParams(use_tc_tiling_on_sc=False,
                                  needs_layout_passes=False)


def _deg_body(dst3_hbm, ones_hbm, zeros_hbm, out_hbm, acc, ones_v, dstw_v,
              drain_v, sem):
    cid = lax.axis_index("c")
    tid = lax.axis_index("s")
    wid = cid * _NS + tid
    pltpu.sync_copy(zeros_hbm, drain_v)
    pltpu.sync_copy(drain_v, acc.at[pl.ds(tid * _RPT, _RPT)])
    pltpu.sync_copy(ones_hbm, ones_v)
    pltpu.sync_copy(dst3_hbm.at[wid], dstw_v)
    plsc.subcore_barrier()

    k = 25
    for grp in range(_NCHUNK // k):
        def fire(j, carry):
            pltpu.async_copy(ones_v, acc.at[dstw_v.at[j]], sem, add=True)
            return carry

        def drain(j, carry):
            pltpu.make_async_copy(ones_v, acc.at[dstw_v.at[j]], sem).wait()
            return carry

        lax.fori_loop(grp * k, (grp + 1) * k, fire, 0)
        lax.fori_loop(grp * k, (grp + 1) * k, drain, 0)

    plsc.subcore_barrier()
    pltpu.sync_copy(acc.at[pl.ds(tid * _RPT, _RPT)], drain_v)
    pltpu.sync_copy(drain_v, out_hbm.at[cid, pl.ds(tid * _RPT, _RPT)])


_deg_call = functools.partial(
    pl.kernel,
    compiler_params=_SC_PARAMS,
    out_type=jax.ShapeDtypeStruct((_NC, _NP, _DEGW), jnp.float32),
    mesh=_sc_mesh(),
    scratch_types=[
        pltpu.VMEM_SHARED((_NP, _DEGW), jnp.float32),
        pltpu.VMEM((_CH, _DEGW), jnp.float32),
        pltpu.VMEM((_NCHUNK, _CH), jnp.int32),
        pltpu.VMEM((_RPT, _DEGW), jnp.float32),
        pltpu.SemaphoreType.DMA,
    ],
)(_deg_body)


def _conv_body(t_hbm, src_hbm, dst3_hbm, zeros_hbm, out_hbm, acc, src_v,
               dstw_v, rows0, rows1, semg0, semg1, sems0, sems1):
    cid = lax.axis_index("c")
    tid = lax.axis_index("s")
    wid = cid * _NS + tid
    pltpu.sync_copy(zeros_hbm, rows0)
    for k in range(_RPT // _CH):
        pltpu.sync_copy(rows0, acc.at[pl.ds(tid * _RPT + k * _CH, _CH)])
    base = wid * _EPW
    pltpu.sync_copy(src_hbm.at[pl.ds(base, _EPW)], src_v)
    pltpu.sync_copy(dst3_hbm.at[wid], dstw_v)
    plsc.subcore_barrier()

    def gstart(j, rows, sem):
        pltpu.async_copy(t_hbm.at[src_v.at[pl.ds(j * _CH, _CH)]], rows, sem)

    def gwait(j, rows, sem):
        pltpu.make_async_copy(t_hbm.at[src_v.at[pl.ds(j * _CH, _CH)]],
                              rows, sem).wait()

    def sstart(j, rows, sem):
        pltpu.async_copy(rows, acc.at[dstw_v.at[j]], sem, add=True)

    def swait(j, rows, sem):
        pltpu.make_async_copy(rows, acc.at[dstw_v.at[j]], sem).wait()

    npair = (_NCHUNK - 1) // 2
    gstart(0, rows0, semg0)

    def pair(m, carry):
        j0 = 2 * m
        gstart(j0 + 1, rows1, semg1)
        gwait(j0, rows0, semg0)
        pltpu.sync_copy(rows0, acc.at[dstw_v.at[j0]], add=True)

        @pl.when(m < npair - 1)
        def _():
            gstart(j0 + 2, rows0, semg0)

        gwait(j0 + 1, rows1, semg1)
        pltpu.sync_copy(rows1, acc.at[dstw_v.at[j0 + 1]], add=True)
        return carry

    lax.fori_loop(0, npair, pair, 0)
    jt = _NCHUNK - 1
    gstart(jt, rows0, semg0)
    gwait(jt, rows0, semg0)
    pltpu.sync_copy(rows0, acc.at[dstw_v.at[jt]], add=True)
    plsc.subcore_barrier()
    for k in range(_RPT // _CH):
        rb = rows0 if k % 2 == 0 else rows1
        pltpu.sync_copy(acc.at[pl.ds(tid * _RPT + k * _CH, _CH)], rb)
        pltpu.sync_copy(rb, out_hbm.at[cid, pl.ds(tid * _RPT + k * _CH, _CH)])


_conv_call = functools.partial(
    pl.kernel,
    compiler_params=_SC_PARAMS,
    out_type=jax.ShapeDtypeStruct((_NC, _NP, D), jnp.float32),
    mesh=_sc_mesh(),
    scratch_types=[
        pltpu.VMEM_SHARED((_NP, D), jnp.float32),
        pltpu.VMEM((_EPW,), jnp.int32),
        pltpu.VMEM((_NCHUNK, _CH), jnp.int32),
        pltpu.VMEM((_CH, D), jnp.float32),
        pltpu.VMEM((_CH, D), jnp.float32),
        pltpu.SemaphoreType.DMA,
        pltpu.SemaphoreType.DMA,
        pltpu.SemaphoreType.DMA,
        pltpu.SemaphoreType.DMA,
    ],
)(_conv_body)


def _pool_body(h_hbm, batch_hbm, ninf_hbm, zsum_hbm, zcnt_hbm,
               sum_out, max_out, cnt_out,
               rbuf, bbuf, sumacc, maxacc, cntacc):
    cid = lax.axis_index("c")
    tid = lax.axis_index("s")
    wid = cid * _NS + tid
    pltpu.sync_copy(ninf_hbm, maxacc)
    pltpu.sync_copy(zsum_hbm, sumacc)
    pltpu.sync_copy(zcnt_hbm, cntacc)

    lane = lax.broadcasted_iota(jnp.int32, (16,), 0)
    lane0 = lane == 0
    zero16 = jnp.zeros((16,), jnp.int32)
    dnums = lax.GatherDimensionNumbers(
        offset_dims=(), collapsed_slice_dims=(0,), start_index_map=(0,))

    def do_chunk(k, carry):
        cidx = wid + _NW * k
        row0 = cidx * _PCH
        pltpu.sync_copy(h_hbm.at[pl.ds(row0, _PCH)], rbuf)
        pltpu.sync_copy(batch_hbm.at[pl.ds(row0, _PCH)], bbuf)

        def do_sub(b, carry2):
            bvec = bbuf[pl.ds(b * 16, 16)]

            def do_lane(l, carry3):
                idx = jnp.full((16, 1), 0, jnp.int32) + l
                g16 = lax.gather(bvec, idx, dnums, (1,),
                                 mode=lax.GatherScatterMode.PROMISE_IN_BOUNDS)
                r = b * 16 + l
                for c in range(D // 16):
                    col = lane + c * 16
                    v = rbuf[r, pl.ds(c * 16, 16)]
                    cur = plsc.load_gather(maxacc, [g16, col])
                    plsc.store_scatter(maxacc, [g16, col], jnp.maximum(cur, v))
                    cur2 = plsc.load_gather(sumacc, [g16, col])
                    plsc.store_scatter(sumacc, [g16, col], cur2 + v)
                cnt = plsc.load_gather(cntacc, [zero16, g16])
                plsc.store_scatter(cntacc, [zero16, g16], cnt + 1.0, mask=lane0)
                return carry3

            return lax.fori_loop(0, 16, do_lane, carry2)

        lax.fori_loop(0, _PCH // 16, do_sub, carry)
        return carry

    nch = jnp.where(wid < _PNC - 3 * _NW, 4, 3)
    lax.fori_loop(0, nch, do_chunk, 0)
    pltpu.sync_copy(sumacc, sum_out.at[wid])
    pltpu.sync_copy(maxacc, max_out.at[wid])
    pltpu.sync_copy(cntacc, cnt_out.at[wid])


_pool_call = functools.partial(
    pl.kernel,
    compiler_params=_SC_PARAMS,
    out_type=(
        jax.ShapeDtypeStruct((_NW, G, D), jnp.float32),
        jax.ShapeDtypeStruct((_NW, G, D), jnp.float32),
        jax.ShapeDtypeStruct((_NW, 8, G), jnp.float32),
    ),
    mesh=_sc_mesh(),
    scratch_types=[
        pltpu.VMEM((_PCH, D), jnp.float32),
        pltpu.VMEM((_PCH,), jnp.int32),
        pltpu.VMEM((G, D), jnp.float32),
        pltpu.VMEM((G, D), jnp.float32),
        pltpu.VMEM((8, G), jnp.float32),
    ],
)(_pool_body)


def _pre1_body(x_ref, degp_ref, w_ref, t_ref, dinv_ref):
    dp = degp_ref[...]
    deg = dp[0, :, 0:1] + dp[1, :, 0:1] + 1.0
    dinv = 1.0 / jnp.sqrt(deg)
    dinv_ref[...] = dinv
    t_ref[...] = jnp.dot(x_ref[...], w_ref[...],
                         preferred_element_type=jnp.float32) * dinv


def _pre1_call(x, degp, w):
    return pl.pallas_call(
        _pre1_body,
        grid=(_GRID,),
        in_specs=[
            pl.BlockSpec((_BLK, D), lambda i: (i, 0)),
            pl.BlockSpec((_NC, _BLK, _DEGW), lambda i: (0, i, 0)),
            pl.BlockSpec((D, D), lambda i: (0, 0)),
        ],
        out_specs=[
            pl.BlockSpec((_BLK, D), lambda i: (i, 0)),
            pl.BlockSpec((_BLK, 1), lambda i: (i, 0)),
        ],
        out_shape=[
            jax.ShapeDtypeStruct((N, D), jnp.float32),
            jax.ShapeDtypeStruct((N, 1), jnp.float32),
        ],
    )(x, degp, w)


def _make_post_body(relu_ln, has_res, has_next):
    def body(*refs):
        refs = list(refs)
        conv_ref = refs.pop(0)
        t_ref = refs.pop(0)
        dinv_ref = refs.pop(0)
        b_ref = refs.pop(0)
        gamma_ref = refs.pop(0) if relu_ln else None
        beta_ref = refs.pop(0) if relu_ln else None
        res_ref = refs.pop(0) if has_res else None
        w_ref = refs.pop(0) if has_next else None
        h_ref = refs.pop(0)
        tn_ref = refs.pop(0) if has_next else None

        dinv = dinv_ref[...]
        s = (conv_ref[0] + conv_ref[1] + t_ref[...]) * dinv + b_ref[...]
        if relu_ln:
            a = jnp.maximum(s, 0.0)
            mu = jnp.mean(a, axis=-1, keepdims=True)
            var = jnp.mean((a - mu) ** 2, axis=-1, keepdims=True)
            h = (a - mu) / jnp.sqrt(var + 1e-5) * gamma_ref[...] + beta_ref[...]
        else:
            h = s
        if has_res:
            h = h + res_ref[...]
        h_ref[...] = h
        if has_next:
            tn_ref[...] = jnp.dot(h, w_ref[...],
                                  preferred_element_type=jnp.float32) * dinv
    return body


def _post_call(convp, t, dinv, b, gamma, beta, res, w_next, relu_ln):
    has_res = res is not None
    has_next = w_next is not None
    in_specs = [
        pl.BlockSpec((_NC, _BLK, D), lambda i: (0, i, 0)),
        pl.BlockSpec((_BLK, D), lambda i: (i, 0)),
        pl.BlockSpec((_BLK, 1), lambda i: (i, 0)),
        pl.BlockSpec((1, D), lambda i: (0, 0)),
    ]
    args = [convp, t, dinv, b.reshape(1, D)]
    if relu_ln:
        in_specs += [pl.BlockSpec((1, D), lambda i: (0, 0))] * 2
        args += [gamma.reshape(1, D), beta.reshape(1, D)]
    if has_res:
        in_specs.append(pl.BlockSpec((_BLK, D), lambda i: (i, 0)))
        args.append(res)
    if has_next:
        in_specs.append(pl.BlockSpec((D, D), lambda i: (0, 0)))
        args.append(w_next)
    out_specs = [pl.BlockSpec((_BLK, D), lambda i: (i, 0))]
    out_shape = [jax.ShapeDtypeStruct((N, D), jnp.float32)]
    if has_next:
        out_specs.append(pl.BlockSpec((_BLK, D), lambda i: (i, 0)))
        out_shape.append(jax.ShapeDtypeStruct((N, D), jnp.float32))
    res_out = pl.pallas_call(
        _make_post_body(relu_ln, has_res, has_next),
        grid=(_GRID,),
        in_specs=in_specs,
        out_specs=out_specs,
        out_shape=out_shape,
    )(*args)
    return res_out if has_next else res_out[0]


def _head_body(sum_ref, max_ref, cnt_ref, w1_ref, b1_ref, w2_ref, b2_ref,
               out_ref):
    cnt = cnt_ref[0, 0]
    for i in range(1, _NW):
        cnt = cnt + cnt_ref[i, 0]
    s = sum_ref[0]
    m = max_ref[0]
    for i in range(1, _NW):
        s = s + sum_ref[i]
        m = jnp.maximum(m, max_ref[i])
    counts = jnp.maximum(cnt, 1.0)[:, None]
    mean = s / counts
    m = jnp.where(m == -jnp.inf, 0.0, m)
    gcat = jnp.concatenate([mean, m], axis=1)
    a = jnp.maximum(
        jnp.dot(gcat, w1_ref[...], preferred_element_type=jnp.float32)
        + b1_ref[...], 0.0)
    out_ref[...] = jnp.dot(a, w2_ref[...],
                           preferred_element_type=jnp.float32) + b2_ref[...]


def _head_call(sump, maxp, cntp, w1, b1, w2, b2):
    nout = w2.shape[1]
    return pl.pallas_call(
        _head_body,
        out_shape=jax.ShapeDtypeStruct((G, nout), jnp.float32),
    )(sump, maxp, cntp, w1, b1.reshape(1, D), w2, b2.reshape(1, nout))


def kernel(x, edge_index, batch, W_in, b_in, W1, b1, W2, b2, W3, b3,
           W_out, b_out, gamma, beta, fc1_W, fc1_b, fc2_W, fc2_b):
    f32 = jnp.float32
    src = edge_index[0]
    dst3 = edge_index[1].reshape(_NW, _NCHUNK, _CH)

    zeros_deg = jnp.zeros((_RPT, _DEGW), f32)
    ones_deg = jnp.ones((_CH, _DEGW), f32)
    zeros_conv = jnp.zeros((_CH, D), f32)
    ninf = jnp.full((G, D), -jnp.inf, f32)
    zsum = jnp.zeros((G, D), f32)
    zcnt = jnp.zeros((8, G), f32)

    degp = _deg_call(dst3, ones_deg, zeros_deg)

    t, dinv = _pre1_call(x, degp, W_in)

    convp = _conv_call(t, src, dst3, zeros_conv)
    h, t = _post_call(convp, t, dinv, b_in, gamma, beta, None, W1,
                      relu_ln=True)
    for (b_cur, w_next) in [(b1, W2), (b2, W3), (b3, W_out)]:
        convp = _conv_call(t, src, dst3, zeros_conv)
        h, t = _post_call(convp, t, dinv, b_cur, gamma, beta, h, w_next,
                          relu_ln=True)
    convp = _conv_call(t, src, dst3, zeros_conv)
    h_out = _post_call(convp, t, dinv, b_out, None, None, None, None,
                       relu_ln=False)

    sump, maxp, cntp = _pool_call(h_out, batch, ninf, zsum, zcnt)

    return _head_call(sump, maxp, cntp, fc1_W, fc1_b, fc2_W, fc2_b)
